# baseline (speedup 1.0000x reference)
import functools

import jax
import jax.numpy as jnp
from jax import lax
from jax.experimental import pallas as pl
from jax.experimental.pallas import tpu as pltpu
from jax.experimental.pallas import tpu_sc as plsc

NUM_EDGES = 320000
DIM = 128
NC = 2
NS = 16
NW = NC * NS
CHUNK = 128
EPW = 10240
NUM_PAD = EPW * NW
NCHUNKS = EPW // CHUNK
LANES = 16
GROUPS = CHUNK // LANES


def _edge_decoder(z_hbm, src_hbm, dst_hbm, out_hbm,
                  sidx_v, didx_v, srows0, drows0, srows1, drows1,
                  part_v, out_v, sem0, sem1):
    wid = lax.axis_index("s") * NC + lax.axis_index("c")
    base = wid * EPW

    lane_iota = lax.iota(jnp.int32, LANES)
    rows_bufs = ((srows0, drows0), (srows1, drows1))
    sems = (sem0, sem1)

    pltpu.sync_copy(src_hbm.at[pl.ds(base, EPW)], sidx_v)
    pltpu.sync_copy(dst_hbm.at[pl.ds(base, EPW)], didx_v)

    def issue(c, buf):
        srows, drows = rows_bufs[buf]
        off = pl.multiple_of(c * CHUNK, 8)
        pltpu.async_copy(z_hbm.at[sidx_v.at[pl.ds(off, CHUNK)]], srows,
                         sems[buf])
        pltpu.async_copy(z_hbm.at[didx_v.at[pl.ds(off, CHUNK)]], drows,
                         sems[buf])

    def wait(buf):
        srows, drows = rows_bufs[buf]
        dummy = z_hbm.at[pl.ds(0, CHUNK)]
        pltpu.make_async_copy(dummy, srows, sems[buf]).wait()
        pltpu.make_async_copy(dummy, drows, sems[buf]).wait()

    def compute(c, buf):
        srows, drows = rows_bufs[buf]

        def group_body(g, carry):
            gbase = g * LANES
            for e in range(LANES):
                row = gbase + e
                p = (srows[row, pl.ds(0, LANES)]
                     * drows[row, pl.ds(0, LANES)])
                for dd in range(1, DIM // LANES):
                    p = p + (srows[row, pl.ds(dd * LANES, LANES)]
                             * drows[row, pl.ds(dd * LANES, LANES)])
                part_v[pl.ds(e * LANES, LANES)] = p

            acc = jnp.zeros((LANES,), jnp.float32)
            for l in range(LANES):
                acc = acc + plsc.load_gather(part_v, [lane_iota * LANES + l])
            obase = pl.multiple_of(c * CHUNK, 8) + gbase
            out_v[pl.ds(obase, LANES)] = 1.0 / (1.0 + jnp.exp(-acc))
            return carry

        lax.fori_loop(0, GROUPS, group_body, 0)

    issue(0, 0)
    issue(1, 1)

    def pair_body(i, carry):
        c0 = i * 2
        wait(0)
        compute(c0, 0)
        issue(c0 + 2, 0)
        wait(1)
        compute(c0 + 1, 1)
        issue(c0 + 3, 1)
        return carry

    lax.fori_loop(0, NCHUNKS // 2 - 1, pair_body, 0)
    wait(0)
    compute(NCHUNKS - 2, 0)
    wait(1)
    compute(NCHUNKS - 1, 1)

    real = NUM_EDGES - (NW - 1) * EPW

    @pl.when(base + EPW <= NUM_EDGES)
    def _full():
        pltpu.sync_copy(out_v, out_hbm.at[pl.ds(base, EPW)])

    @pl.when(base + EPW > NUM_EDGES)
    def _tail():
        pltpu.sync_copy(out_v.at[pl.ds(0, real)],
                        out_hbm.at[pl.ds(base, real)])


@jax.jit
def _run(z, src, dst):
    mesh = plsc.VectorSubcoreMesh(core_axis_name="c", subcore_axis_name="s")
    return pl.kernel(
        _edge_decoder,
        out_type=jax.ShapeDtypeStruct((NUM_EDGES,), jnp.float32),
        mesh=mesh,
        scratch_types=[
            pltpu.VMEM((EPW,), jnp.int32),
            pltpu.VMEM((EPW,), jnp.int32),
            pltpu.VMEM((CHUNK, DIM), jnp.float32),
            pltpu.VMEM((CHUNK, DIM), jnp.float32),
            pltpu.VMEM((CHUNK, DIM), jnp.float32),
            pltpu.VMEM((CHUNK, DIM), jnp.float32),
            pltpu.VMEM((LANES * LANES,), jnp.float32),
            pltpu.VMEM((EPW,), jnp.float32),
            pltpu.SemaphoreType.DMA,
            pltpu.SemaphoreType.DMA,
        ],
        compiler_params=pltpu.CompilerParams(needs_layout_passes=False),
    )(z, src, dst)


def kernel(z, edge_index):
    edge_index = edge_index.astype(jnp.int32)
    pad = jnp.zeros((NUM_PAD - NUM_EDGES,), jnp.int32)
    src = jnp.concatenate([edge_index[0], pad])
    dst = jnp.concatenate([edge_index[1], pad])
    return _run(z, src, dst)
